# baseline (speedup 1.0000x reference)
import functools

import jax
import jax.numpy as jnp
from jax import lax
from jax.experimental import pallas as pl
from jax.experimental.pallas import tpu as pltpu
from jax.experimental.pallas import tpu_sc as plsc


def _make_gather(vocab, d, b):
    info = plsc.get_sparse_core_info()
    nc, ns = info.num_cores, info.num_subcores
    nw = nc * ns
    chunk = 1024
    sub = chunk // 128
    per_w = b // nw
    steps = per_w // chunk
    assert b % (nw * chunk) == 0 and d % 16 == 0

    mesh = plsc.VectorSubcoreMesh(core_axis_name="c", subcore_axis_name="s")

    @functools.partial(
        pl.kernel,
        mesh=mesh,
        out_type=jax.ShapeDtypeStruct((b, d), jnp.float32),
        compiler_params=pltpu.CompilerParams(use_tc_tiling_on_sc=False),
        scratch_types=[
            pltpu.VMEM((sub, 128), jnp.int32),
            pltpu.VMEM((chunk, d), jnp.float32),
            pltpu.SemaphoreType.DMA,
        ],
    )
    def gather(table_hbm, idx_hbm, out_hbm, idx_v, rows_v, sem):
        wid = lax.axis_index("s") * nc + lax.axis_index("c")

        def step(i, carry):
            row0 = wid * (per_w // 128) + i * sub
            off = wid * per_w + i * chunk
            pltpu.sync_copy(idx_hbm.at[pl.ds(row0, sub)], idx_v)
            copies = [
                pltpu.async_copy(
                    table_hbm.at[idx_v.at[j]],
                    rows_v.at[pl.ds(j * 128, 128)],
                    sem,
                )
                for j in range(sub)
            ]
            for c in copies:
                c.wait()
            pltpu.sync_copy(rows_v, out_hbm.at[pl.ds(off, chunk)])
            return carry

        lax.fori_loop(0, steps, step, 0)

    return gather


def kernel(text, table):
    bsz, seq = text.shape
    vocab, d = table.shape
    b = bsz * seq
    idx = text.reshape(b // 128, 128).astype(jnp.int32)
    out = _make_gather(vocab, d, b)(table, idx)
    return out.reshape(bsz, seq, d)


# speedup vs baseline: 1.0061x; 1.0061x over previous
import functools

import jax
import jax.numpy as jnp
from jax import lax
from jax.experimental import pallas as pl
from jax.experimental.pallas import tpu as pltpu
from jax.experimental.pallas import tpu_sc as plsc


def _make_gather(vocab, d, b):
    info = plsc.get_sparse_core_info()
    nc, ns = info.num_cores, info.num_subcores
    nw = nc * ns
    chunk = 512
    sub = chunk // 128
    per_w = b // nw
    steps = per_w // chunk
    assert b % (nw * chunk) == 0 and d % 16 == 0
    assert steps >= 2 and steps % 2 == 0

    mesh = plsc.VectorSubcoreMesh(core_axis_name="c", subcore_axis_name="s")

    @functools.partial(
        pl.kernel,
        mesh=mesh,
        out_type=jax.ShapeDtypeStruct((b, d), jnp.float32),
        compiler_params=pltpu.CompilerParams(use_tc_tiling_on_sc=False),
        scratch_types=[
            pltpu.VMEM((2, sub, 128), jnp.int32),
            pltpu.VMEM((2, chunk, d), jnp.float32),
            pltpu.SemaphoreType.DMA,
            pltpu.SemaphoreType.DMA,
            pltpu.SemaphoreType.DMA,
            pltpu.SemaphoreType.DMA,
        ],
    )
    def gather(table_hbm, idx_hbm, out_hbm, idx_v, rows_v, sg0, sg1, so0, so1):
        wid = lax.axis_index("s") * nc + lax.axis_index("c")
        row_base = wid * (per_w // 128)
        out_base = wid * per_w
        sg = (sg0, sg1)
        so = (so0, so1)

        def load_and_fire(c, bb):
            pltpu.sync_copy(idx_hbm.at[pl.ds(row_base + c * sub, sub)],
                            idx_v.at[bb])
            for j in range(sub):
                pltpu.async_copy(
                    table_hbm.at[idx_v.at[bb].at[j]],
                    rows_v.at[bb].at[pl.ds(j * 128, 128)],
                    sg[bb],
                )

        def drain_gathers(bb):
            pltpu.make_async_copy(out_hbm.at[pl.ds(0, chunk)],
                                  rows_v.at[bb], sg[bb]).wait()

        def fire_store(c, bb):
            pltpu.async_copy(rows_v.at[bb],
                             out_hbm.at[pl.ds(out_base + c * chunk, chunk)],
                             so[bb])

        def drain_store(bb):
            pltpu.make_async_copy(rows_v.at[bb],
                                  out_hbm.at[pl.ds(0, chunk)], so[bb]).wait()

        load_and_fire(0, 0)
        load_and_fire(1, 1)
        drain_gathers(0)
        fire_store(0, 0)

        def outer(i, carry):
            for bb in (0, 1):
                c = 2 + i * 2 + bb
                pb = bb ^ 1
                drain_store(bb)
                load_and_fire(c, bb)
                drain_gathers(pb)
                fire_store(c - 1, pb)
            return carry

        lax.fori_loop(0, (steps - 2) // 2, outer, 0)

        drain_gathers(1)
        fire_store(steps - 1, 1)
        drain_store(0)
        drain_store(1)

    return gather


def kernel(text, table):
    bsz, seq = text.shape
    vocab, d = table.shape
    b = bsz * seq
    idx = text.reshape(b // 128, 128).astype(jnp.int32)
    out = _make_gather(vocab, d, b)(table, idx)
    return out.reshape(bsz, seq, d)


# speedup vs baseline: 1.0064x; 1.0003x over previous
import functools

import jax
import jax.numpy as jnp
from jax import lax
from jax.experimental import pallas as pl
from jax.experimental.pallas import tpu as pltpu
from jax.experimental.pallas import tpu_sc as plsc


def _make_gather(vocab, d, b):
    info = plsc.get_sparse_core_info()
    nc, ns = info.num_cores, info.num_subcores
    nw = nc * ns
    chunk = 512
    sub = chunk // 128
    per_w = b // nw
    steps = per_w // chunk
    assert b % (nw * chunk) == 0 and d % 16 == 0
    assert steps >= 2 and steps % 2 == 0

    mesh = plsc.VectorSubcoreMesh(core_axis_name="c", subcore_axis_name="s")

    @functools.partial(
        pl.kernel,
        mesh=mesh,
        out_type=jax.ShapeDtypeStruct((b, d), jnp.float32),
        compiler_params=pltpu.CompilerParams(use_tc_tiling_on_sc=False),
        scratch_types=[
            pltpu.VMEM((chunk,), jnp.int32),
            pltpu.VMEM((chunk,), jnp.int32),
            pltpu.VMEM((2, chunk, d), jnp.float32),
            pltpu.SemaphoreType.DMA,
            pltpu.SemaphoreType.DMA,
            pltpu.SemaphoreType.DMA,
            pltpu.SemaphoreType.DMA,
        ],
    )
    def gather(table_hbm, idx_hbm, out_hbm, idx_v0, idx_v1, rows_v,
               sg0, sg1, so0, so1):
        wid = lax.axis_index("s") * nc + lax.axis_index("c")
        out_base = wid * per_w
        idx_v = (idx_v0, idx_v1)
        sg = (sg0, sg1)
        so = (so0, so1)

        def load_and_fire(c, bb):
            pltpu.sync_copy(idx_hbm.at[pl.ds(out_base + c * chunk, chunk)],
                            idx_v[bb])
            pltpu.async_copy(table_hbm.at[idx_v[bb]], rows_v.at[bb], sg[bb])

        def drain_gathers(bb):
            pltpu.make_async_copy(out_hbm.at[pl.ds(0, chunk)],
                                  rows_v.at[bb], sg[bb]).wait()

        def fire_store(c, bb):
            pltpu.async_copy(rows_v.at[bb],
                             out_hbm.at[pl.ds(out_base + c * chunk, chunk)],
                             so[bb])

        def drain_store(bb):
            pltpu.make_async_copy(rows_v.at[bb],
                                  out_hbm.at[pl.ds(0, chunk)], so[bb]).wait()

        load_and_fire(0, 0)
        load_and_fire(1, 1)
        drain_gathers(0)
        fire_store(0, 0)

        def outer(i, carry):
            for bb in (0, 1):
                c = 2 + i * 2 + bb
                pb = bb ^ 1
                drain_store(bb)
                load_and_fire(c, bb)
                drain_gathers(pb)
                fire_store(c - 1, pb)
            return carry

        lax.fori_loop(0, (steps - 2) // 2, outer, 0)

        drain_gathers(1)
        fire_store(steps - 1, 1)
        drain_store(0)
        drain_store(1)

    return gather


def kernel(text, table):
    bsz, seq = text.shape
    vocab, d = table.shape
    b = bsz * seq
    idx = text.reshape(b).astype(jnp.int32)
    out = _make_gather(vocab, d, b)(table, idx)
    return out.reshape(bsz, seq, d)


# speedup vs baseline: 1.0087x; 1.0023x over previous
import functools

import jax
import jax.numpy as jnp
from jax import lax
from jax.experimental import pallas as pl
from jax.experimental.pallas import tpu as pltpu
from jax.experimental.pallas import tpu_sc as plsc


def _make_gather(vocab, d, bsz, seq):
    info = plsc.get_sparse_core_info()
    nc, ns = info.num_cores, info.num_subcores
    nw = nc * ns
    rows_w = bsz // nw
    r = 4
    steps = rows_w // r
    assert bsz % nw == 0 and rows_w % r == 0
    assert steps >= 2 and steps % 2 == 0

    mesh = plsc.VectorSubcoreMesh(core_axis_name="c", subcore_axis_name="s")

    @functools.partial(
        pl.kernel,
        mesh=mesh,
        out_type=jax.ShapeDtypeStruct((bsz, seq, d), jnp.float32),
        compiler_params=pltpu.CompilerParams(use_tc_tiling_on_sc=False),
        scratch_types=[
            pltpu.VMEM((2, r, seq), jnp.int32),
            pltpu.VMEM((2, r, seq, d), jnp.float32),
            pltpu.SemaphoreType.DMA,
            pltpu.SemaphoreType.DMA,
            pltpu.SemaphoreType.DMA,
            pltpu.SemaphoreType.DMA,
        ],
    )
    def gather(table_hbm, text_hbm, out_hbm, idx_v, rows_v, sg0, sg1, so0, so1):
        wid = lax.axis_index("s") * nc + lax.axis_index("c")
        row_base = wid * rows_w
        sg = (sg0, sg1)
        so = (so0, so1)

        def load_and_fire(c, bb):
            pltpu.sync_copy(text_hbm.at[pl.ds(row_base + c * r, r)],
                            idx_v.at[bb])
            for j in range(r):
                pltpu.async_copy(
                    table_hbm.at[idx_v.at[bb].at[j]],
                    rows_v.at[bb].at[j],
                    sg[bb],
                )

        def drain_gathers(bb):
            pltpu.make_async_copy(out_hbm.at[pl.ds(0, r)],
                                  rows_v.at[bb], sg[bb]).wait()

        def fire_store(c, bb):
            pltpu.async_copy(rows_v.at[bb],
                             out_hbm.at[pl.ds(row_base + c * r, r)],
                             so[bb])

        def drain_store(bb):
            pltpu.make_async_copy(rows_v.at[bb],
                                  out_hbm.at[pl.ds(0, r)], so[bb]).wait()

        load_and_fire(0, 0)
        load_and_fire(1, 1)
        drain_gathers(0)
        fire_store(0, 0)

        def outer(i, carry):
            for bb in (0, 1):
                c = 2 + i * 2 + bb
                pb = bb ^ 1
                drain_store(bb)
                load_and_fire(c, bb)
                drain_gathers(pb)
                fire_store(c - 1, pb)
            return carry

        lax.fori_loop(0, (steps - 2) // 2, outer, 0)

        drain_gathers(1)
        fire_store(steps - 1, 1)
        drain_store(0)
        drain_store(1)

    return gather


def kernel(text, table):
    bsz, seq = text.shape
    vocab, d = table.shape
    return _make_gather(vocab, d, bsz, seq)(table, text.astype(jnp.int32))


# speedup vs baseline: 1.2402x; 1.2294x over previous
import functools

import jax
import jax.numpy as jnp
from jax import lax
from jax.experimental import pallas as pl
from jax.experimental.pallas import tpu as pltpu
from jax.experimental.pallas import tpu_sc as plsc


def _make_gather(vocab, d, bsz, seq):
    info = plsc.get_sparse_core_info()
    nc, ns = info.num_cores, info.num_subcores
    nw = nc * ns
    rows_w = bsz // nw
    r = 2
    steps = rows_w // r
    assert bsz % nw == 0 and rows_w % r == 0
    assert steps >= 2 and steps % 2 == 0

    mesh = plsc.VectorSubcoreMesh(core_axis_name="c", subcore_axis_name="s")

    @functools.partial(
        pl.kernel,
        mesh=mesh,
        out_type=jax.ShapeDtypeStruct((bsz, seq, d), jnp.float32),
        compiler_params=pltpu.CompilerParams(use_tc_tiling_on_sc=False),
        scratch_types=[
            pltpu.VMEM((2, r, seq), jnp.int32),
            pltpu.VMEM((2, r, seq, d), jnp.float32),
            pltpu.SemaphoreType.DMA,
            pltpu.SemaphoreType.DMA,
            pltpu.SemaphoreType.DMA,
            pltpu.SemaphoreType.DMA,
        ],
    )
    def gather(table_hbm, text_hbm, out_hbm, idx_v, rows_v, sg0, sg1, so0, so1):
        wid = lax.axis_index("s") * nc + lax.axis_index("c")
        row_base = wid * rows_w
        sg = (sg0, sg1)
        so = (so0, so1)

        def load_and_fire(c, bb):
            pltpu.sync_copy(text_hbm.at[pl.ds(row_base + c * r, r)],
                            idx_v.at[bb])
            for j in range(r):
                pltpu.async_copy(
                    table_hbm.at[idx_v.at[bb].at[j]],
                    rows_v.at[bb].at[j],
                    sg[bb],
                )

        def drain_gathers(bb):
            pltpu.make_async_copy(out_hbm.at[pl.ds(0, r)],
                                  rows_v.at[bb], sg[bb]).wait()

        def fire_store(c, bb):
            pltpu.async_copy(rows_v.at[bb],
                             out_hbm.at[pl.ds(row_base + c * r, r)],
                             so[bb])

        def drain_store(bb):
            pltpu.make_async_copy(rows_v.at[bb],
                                  out_hbm.at[pl.ds(0, r)], so[bb]).wait()

        load_and_fire(0, 0)
        load_and_fire(1, 1)
        drain_gathers(0)
        fire_store(0, 0)

        def outer(i, carry):
            for bb in (0, 1):
                c = 2 + i * 2 + bb
                pb = bb ^ 1
                drain_store(bb)
                load_and_fire(c, bb)
                drain_gathers(pb)
                fire_store(c - 1, pb)
            return carry

        lax.fori_loop(0, (steps - 2) // 2, outer, 0)

        drain_gathers(1)
        fire_store(steps - 1, 1)
        drain_store(0)
        drain_store(1)

    return gather


def kernel(text, table):
    bsz, seq = text.shape
    vocab, d = table.shape
    dp = 128
    tp = jnp.pad(table, ((0, 0), (0, dp - d)))
    out_pad = _make_gather(vocab, dp, bsz, seq)(tp, text.astype(jnp.int32))
    return out_pad[..., :d]


# speedup vs baseline: 1.5221x; 1.2273x over previous
import functools

import jax
import jax.numpy as jnp
from jax import lax
from jax.experimental import pallas as pl
from jax.experimental.pallas import tpu as pltpu
from jax.experimental.pallas import tpu_sc as plsc


def _make_gather(vocab, d, bsz, seq):
    info = plsc.get_sparse_core_info()
    nc, ns = info.num_cores, info.num_subcores
    nw = nc * ns
    rows_w = bsz // nw
    r = 2
    steps = rows_w // r
    assert bsz % nw == 0 and rows_w % r == 0
    assert steps >= 2 and steps % 2 == 0

    mesh = plsc.VectorSubcoreMesh(core_axis_name="c", subcore_axis_name="s")

    @functools.partial(
        pl.kernel,
        mesh=mesh,
        out_type=jax.ShapeDtypeStruct((bsz, seq, d), jnp.float32),
        compiler_params=pltpu.CompilerParams(use_tc_tiling_on_sc=False),
        scratch_types=[
            pltpu.VMEM((2, r, seq), jnp.int32),
            pltpu.VMEM((2, r, seq, d), jnp.float32),
            pltpu.SemaphoreType.DMA,
            pltpu.SemaphoreType.DMA,
            pltpu.SemaphoreType.DMA,
            pltpu.SemaphoreType.DMA,
        ],
    )
    def gather(table_hbm, text_hbm, out_hbm, idx_v, rows_v, sg0, sg1, so0, so1):
        wid = lax.axis_index("s") * nc + lax.axis_index("c")
        row_base = wid * rows_w
        sg = (sg0, sg1)
        so = (so0, so1)

        def load_and_fire(c, bb):
            pltpu.sync_copy(text_hbm.at[pl.ds(row_base + c * r, r)],
                            idx_v.at[bb])
            for j in range(r):
                pltpu.async_copy(
                    table_hbm.at[idx_v.at[bb].at[j]],
                    rows_v.at[bb].at[j],
                    sg[bb],
                )

        def drain_gathers(bb):
            pltpu.make_async_copy(out_hbm.at[pl.ds(0, r)],
                                  rows_v.at[bb], sg[bb]).wait()

        def fire_store(c, bb):
            pltpu.async_copy(rows_v.at[bb],
                             out_hbm.at[pl.ds(row_base + c * r, r)],
                             so[bb])

        def drain_store(bb):
            pltpu.make_async_copy(rows_v.at[bb],
                                  out_hbm.at[pl.ds(0, r)], so[bb]).wait()

        load_and_fire(0, 0)
        load_and_fire(1, 1)
        drain_gathers(0)
        fire_store(0, 0)

        def outer(i, carry):
            for bb in (0, 1):
                c = 2 + i * 2 + bb
                pb = bb ^ 1
                drain_store(bb)
                load_and_fire(c, bb)
                drain_gathers(pb)
                fire_store(c - 1, pb)
            return carry

        lax.fori_loop(0, (steps - 2) // 2, outer, 0)

        drain_gathers(1)
        fire_store(steps - 1, 1)
        drain_store(0)
        drain_store(1)

    return gather


def _transpose_pad(tt, vocab, d, dp):
    w = 4096
    grid = (vocab + w - 1) // w

    def body(t_ref, o_ref):
        blk = t_ref[...].T
        o_ref[...] = jnp.concatenate(
            [blk, jnp.zeros((w, dp - d), jnp.float32)], axis=1)

    return pl.pallas_call(
        body,
        grid=(grid,),
        in_specs=[pl.BlockSpec((d, w), lambda i: (0, i))],
        out_specs=pl.BlockSpec((w, dp), lambda i: (i, 0)),
        out_shape=jax.ShapeDtypeStruct((vocab, dp), jnp.float32),
    )(tt)


def kernel(text, table):
    bsz, seq = text.shape
    vocab, d = table.shape
    dp = 128
    tp = _transpose_pad(table.T, vocab, d, dp)
    out_pad = _make_gather(vocab, dp, bsz, seq)(tp, text.astype(jnp.int32))
    return out_pad[..., :d]


# speedup vs baseline: 1.6666x; 1.0949x over previous
import functools

import jax
import jax.numpy as jnp
from jax import lax
from jax.experimental import pallas as pl
from jax.experimental.pallas import tpu as pltpu
from jax.experimental.pallas import tpu_sc as plsc


def _make_gather(vocab, d, bsz, seq):
    info = plsc.get_sparse_core_info()
    nc, ns = info.num_cores, info.num_subcores
    nw = nc * ns
    rows_w = bsz // nw
    r = 2
    steps = rows_w // r
    assert bsz % nw == 0 and rows_w % r == 0
    assert steps >= 2 and steps % 2 == 0

    mesh = plsc.VectorSubcoreMesh(core_axis_name="c", subcore_axis_name="s")

    @functools.partial(
        pl.kernel,
        mesh=mesh,
        out_type=jax.ShapeDtypeStruct((bsz, seq, d), jnp.float32),
        compiler_params=pltpu.CompilerParams(use_tc_tiling_on_sc=False),
        scratch_types=[
            pltpu.VMEM((2, r, seq), jnp.int32),
            pltpu.VMEM((2, r, seq, d), jnp.float32),
            pltpu.SemaphoreType.DMA,
            pltpu.SemaphoreType.DMA,
            pltpu.SemaphoreType.DMA,
            pltpu.SemaphoreType.DMA,
        ],
    )
    def gather(table_hbm, text_hbm, out_hbm, idx_v, rows_v, sg0, sg1, so0, so1):
        wid = lax.axis_index("s") * nc + lax.axis_index("c")
        row_base = wid * rows_w
        sg = (sg0, sg1)
        so = (so0, so1)

        def load_and_fire(c, bb):
            pltpu.sync_copy(text_hbm.at[pl.ds(row_base + c * r, r)],
                            idx_v.at[bb])
            for j in range(r):
                pltpu.async_copy(
                    table_hbm.at[idx_v.at[bb].at[j]],
                    rows_v.at[bb].at[j],
                    sg[bb],
                )

        def drain_gathers(bb):
            pltpu.make_async_copy(out_hbm.at[pl.ds(0, r)],
                                  rows_v.at[bb], sg[bb]).wait()

        def fire_store(c, bb):
            pltpu.async_copy(rows_v.at[bb],
                             out_hbm.at[pl.ds(row_base + c * r, r)],
                             so[bb])

        def drain_store(bb):
            pltpu.make_async_copy(rows_v.at[bb],
                                  out_hbm.at[pl.ds(0, r)], so[bb]).wait()

        load_and_fire(0, 0)
        load_and_fire(1, 1)
        drain_gathers(0)
        fire_store(0, 0)

        def outer(i, carry):
            for bb in (0, 1):
                c = 2 + i * 2 + bb
                pb = bb ^ 1
                drain_store(bb)
                load_and_fire(c, bb)
                drain_gathers(pb)
                fire_store(c - 1, pb)
            return carry

        lax.fori_loop(0, (steps - 2) // 2, outer, 0)

        drain_gathers(1)
        fire_store(steps - 1, 1)
        drain_store(0)
        drain_store(1)

    return gather


def _transpose_pad(tt, vocab, d, dp):
    w = 8192
    grid = (vocab + w - 1) // w

    def body(t_ref, o_ref):
        blk = t_ref[...].T
        o_ref[...] = jnp.concatenate(
            [blk, jnp.zeros((w, dp - d), jnp.float32)], axis=1)

    return pl.pallas_call(
        body,
        grid=(grid,),
        in_specs=[pl.BlockSpec((d, w), lambda i: (0, i))],
        out_specs=pl.BlockSpec((w, dp), lambda i: (i, 0)),
        out_shape=jax.ShapeDtypeStruct((vocab, dp), jnp.float32),
    )(tt)


def kernel(text, table):
    bsz, seq = text.shape
    vocab, d = table.shape
    dp = 128
    tp = _transpose_pad(table.T, vocab, d, dp)
    out_pad = _make_gather(vocab, dp, bsz, seq)(tp, text.astype(jnp.int32))
    return out_pad[..., :d]


# speedup vs baseline: 1.7108x; 1.0265x over previous
import functools

import jax
import jax.numpy as jnp
from jax import lax
from jax.experimental import pallas as pl
from jax.experimental.pallas import tpu as pltpu
from jax.experimental.pallas import tpu_sc as plsc


def _make_gather(vocab, d, bsz, seq):
    info = plsc.get_sparse_core_info()
    nc, ns = info.num_cores, info.num_subcores
    nw = nc * ns
    rows_w = bsz // nw
    r = 2
    steps = rows_w // r
    assert bsz % nw == 0 and rows_w % r == 0
    assert steps >= 2 and steps % 2 == 0

    mesh = plsc.VectorSubcoreMesh(core_axis_name="c", subcore_axis_name="s")

    @functools.partial(
        pl.kernel,
        mesh=mesh,
        out_type=jax.ShapeDtypeStruct((bsz, seq, d), jnp.float32),
        compiler_params=pltpu.CompilerParams(use_tc_tiling_on_sc=False),
        scratch_types=[
            pltpu.VMEM((2, r, seq), jnp.int32),
            pltpu.VMEM((2, r, seq, d), jnp.float32),
            pltpu.SemaphoreType.DMA,
            pltpu.SemaphoreType.DMA,
            pltpu.SemaphoreType.DMA,
            pltpu.SemaphoreType.DMA,
        ],
    )
    def gather(table_hbm, text_hbm, out_hbm, idx_v, rows_v, sg0, sg1, so0, so1):
        wid = lax.axis_index("s") * nc + lax.axis_index("c")
        row_base = wid * rows_w
        sg = (sg0, sg1)
        so = (so0, so1)

        def load_and_fire(c, bb):
            pltpu.sync_copy(text_hbm.at[pl.ds(row_base + c * r, r)],
                            idx_v.at[bb])
            for j in range(r):
                pltpu.async_copy(
                    table_hbm.at[idx_v.at[bb].at[j]],
                    rows_v.at[bb].at[j],
                    sg[bb],
                )

        def drain_gathers(bb):
            pltpu.make_async_copy(out_hbm.at[pl.ds(0, r)],
                                  rows_v.at[bb], sg[bb]).wait()

        def fire_store(c, bb):
            pltpu.async_copy(rows_v.at[bb],
                             out_hbm.at[pl.ds(row_base + c * r, r)],
                             so[bb])

        def drain_store(bb):
            pltpu.make_async_copy(rows_v.at[bb],
                                  out_hbm.at[pl.ds(0, r)], so[bb]).wait()

        load_and_fire(0, 0)
        load_and_fire(1, 1)
        drain_gathers(0)
        fire_store(0, 0)

        def outer(i, carry):
            for bb in (0, 1):
                c = 2 + i * 2 + bb
                pb = bb ^ 1
                drain_store(bb)
                load_and_fire(c, bb)
                drain_gathers(pb)
                fire_store(c - 1, pb)
            return carry

        lax.fori_loop(0, (steps - 2) // 2, outer, 0)

        drain_gathers(1)
        fire_store(steps - 1, 1)
        drain_store(0)
        drain_store(1)

    return gather


def _transpose_pad(tt, vocab, d, dp):
    w = 16384
    grid = (vocab + w - 1) // w

    def body(t_ref, o_ref):
        blk = t_ref[...].T
        o_ref[...] = jnp.concatenate(
            [blk, jnp.zeros((w, dp - d), jnp.float32)], axis=1)

    return pl.pallas_call(
        body,
        grid=(grid,),
        in_specs=[pl.BlockSpec((d, w), lambda i: (0, i))],
        out_specs=pl.BlockSpec((w, dp), lambda i: (i, 0)),
        out_shape=jax.ShapeDtypeStruct((vocab, dp), jnp.float32),
    )(tt)


def kernel(text, table):
    bsz, seq = text.shape
    vocab, d = table.shape
    dp = 128
    tp = _transpose_pad(table.T, vocab, d, dp)
    out_pad = _make_gather(vocab, dp, bsz, seq)(tp, text.astype(jnp.int32))
    return out_pad[..., :d]


# speedup vs baseline: 1.7239x; 1.0077x over previous
import functools

import jax
import jax.numpy as jnp
from jax import lax
from jax.experimental import pallas as pl
from jax.experimental.pallas import tpu as pltpu
from jax.experimental.pallas import tpu_sc as plsc


def _make_gather(vocab, d, bsz, seq):
    info = plsc.get_sparse_core_info()
    nc, ns = info.num_cores, info.num_subcores
    nw = nc * ns
    rows_w = bsz // nw
    r = 2
    steps = rows_w // r
    assert bsz % nw == 0 and rows_w % r == 0
    assert steps >= 2 and steps % 2 == 0

    mesh = plsc.VectorSubcoreMesh(core_axis_name="c", subcore_axis_name="s")

    @functools.partial(
        pl.kernel,
        mesh=mesh,
        out_type=jax.ShapeDtypeStruct((bsz, seq, d), jnp.float32),
        compiler_params=pltpu.CompilerParams(use_tc_tiling_on_sc=False),
        scratch_types=[
            pltpu.VMEM((2, r, seq), jnp.int32),
            pltpu.VMEM((2, r, seq, d), jnp.float32),
            pltpu.SemaphoreType.DMA,
            pltpu.SemaphoreType.DMA,
            pltpu.SemaphoreType.DMA,
            pltpu.SemaphoreType.DMA,
        ],
    )
    def gather(table_hbm, text_hbm, out_hbm, idx_v, rows_v, sg0, sg1, so0, so1):
        wid = lax.axis_index("s") * nc + lax.axis_index("c")
        row_base = wid * rows_w
        sg = (sg0, sg1)
        so = (so0, so1)

        def load_and_fire(c, bb):
            pltpu.sync_copy(text_hbm.at[pl.ds(row_base + c * r, r)],
                            idx_v.at[bb])
            for j in range(r):
                pltpu.async_copy(
                    table_hbm.at[idx_v.at[bb].at[j]],
                    rows_v.at[bb].at[j],
                    sg[bb],
                )

        def drain_gathers(bb):
            pltpu.make_async_copy(out_hbm.at[pl.ds(0, r)],
                                  rows_v.at[bb], sg[bb]).wait()

        def fire_store(c, bb):
            pltpu.async_copy(rows_v.at[bb],
                             out_hbm.at[pl.ds(row_base + c * r, r)],
                             so[bb])

        def drain_store(bb):
            pltpu.make_async_copy(rows_v.at[bb],
                                  out_hbm.at[pl.ds(0, r)], so[bb]).wait()

        load_and_fire(0, 0)
        load_and_fire(1, 1)
        drain_gathers(0)
        fire_store(0, 0)

        def outer(i, carry):
            for bb in (0, 1):
                c = 2 + i * 2 + bb
                pb = bb ^ 1
                drain_store(bb)
                load_and_fire(c, bb)
                drain_gathers(pb)
                fire_store(c - 1, pb)
            return carry

        lax.fori_loop(0, (steps - 2) // 2, outer, 0)

        drain_gathers(1)
        fire_store(steps - 1, 1)
        drain_store(0)
        drain_store(1)

    return gather


def _transpose_pad(tt, vocab, d, dp):
    w = 32768
    grid = (vocab + w - 1) // w

    def body(t_ref, o_ref):
        blk = t_ref[...].T
        o_ref[...] = jnp.concatenate(
            [blk, jnp.zeros((w, dp - d), jnp.float32)], axis=1)

    return pl.pallas_call(
        body,
        grid=(grid,),
        in_specs=[pl.BlockSpec((d, w), lambda i: (0, i))],
        out_specs=pl.BlockSpec((w, dp), lambda i: (i, 0)),
        out_shape=jax.ShapeDtypeStruct((vocab, dp), jnp.float32),
    )(tt)


def kernel(text, table):
    bsz, seq = text.shape
    vocab, d = table.shape
    dp = 128
    tp = _transpose_pad(table.T, vocab, d, dp)
    out_pad = _make_gather(vocab, dp, bsz, seq)(tp, text.astype(jnp.int32))
    return out_pad[..., :d]
